# baseline (speedup 1.0000x reference)
import functools

import jax
import jax.numpy as jnp
from jax import lax
from jax.experimental import pallas as pl
from jax.experimental.pallas import tpu as pltpu
from jax.experimental.pallas import tpu_sc as plsc

V = 10000
E = 320000
C = 256
HW = 196
HID = 128

NC = 2
NS = 16
NW = NC * NS
EPW = E // NW
CH = 40
NCHUNK = EPW // CH
ND = 4
DCH = 40
ZROWS = 640


def _f32dot(a, b):
    return jnp.dot(a, b, preferred_element_type=jnp.float32)


def _sc_body(nbr_hbm, idx_hbm, out_hbm, agg_sh,
             ibufs, rowsA, rowsB, isems, gsems, ssems):
    c = lax.axis_index("c")
    s = lax.axis_index("s")
    wid = s * NC + c

    def fire_idx(u, b):
        pltpu.async_copy(idx_hbm.at[wid, u], ibufs[b], isems[b])

    def wait_idx(u, b):
        pltpu.make_async_copy(idx_hbm.at[wid, u], ibufs[b], isems[b]).wait()

    def fire_gather(b):
        pltpu.async_copy(nbr_hbm.at[ibufs[b].at[0]], rowsA[b], gsems[b])
        pltpu.async_copy(nbr_hbm.at[ibufs[b].at[1]], rowsB[b], gsems[b])

    def wait_gather(b):
        pltpu.make_async_copy(nbr_hbm.at[ibufs[b].at[0]], rowsA[b], gsems[b]).wait()
        pltpu.make_async_copy(nbr_hbm.at[ibufs[b].at[1]], rowsB[b], gsems[b]).wait()

    def fire_scatter(b):
        pltpu.async_copy(rowsA[b], agg_sh.at[ibufs[b].at[1]], ssems[b], add=True)
        pltpu.async_copy(rowsB[b], agg_sh.at[ibufs[b].at[0]], ssems[b], add=True)

    def wait_scatter(b):
        pltpu.make_async_copy(rowsA[b], agg_sh.at[ibufs[b].at[1]], ssems[b]).wait()
        pltpu.make_async_copy(rowsB[b], agg_sh.at[ibufs[b].at[0]], ssems[b]).wait()

    zvec = jnp.zeros((16,), jnp.float32)
    zbuf = rowsA[0]
    for i in range(DCH):
        for k in range(HID // 16):
            zbuf[i, pl.ds(k * 16, 16)] = zvec

    row0 = pl.multiple_of(s * ZROWS, ZROWS)
    for j in range(ZROWS // DCH):
        start = pl.multiple_of(row0 + j * DCH, DCH)

        @pl.when(start < V)
        def _():
            pltpu.async_copy(zbuf, agg_sh.at[pl.ds(start, DCH)], isems[0])

    for j in range(ZROWS // DCH):
        start = pl.multiple_of(row0 + j * DCH, DCH)

        @pl.when(start < V)
        def _():
            pltpu.make_async_copy(zbuf, agg_sh.at[pl.ds(start, DCH)], isems[0]).wait()

    for b in range(ND):
        pltpu.sync_copy(idx_hbm.at[wid, b], ibufs[b])
        fire_gather(b)

    plsc.subcore_barrier()

    def _quad(k, _):
        c0 = ND * k
        for b in range(ND):
            wait_gather(b)
            fire_scatter(b)
        for b in range(ND):
            wait_scatter(b)

            @pl.when(c0 + b + ND < NCHUNK)
            def _():
                fire_idx(c0 + b + ND, b)
        for b in range(ND):
            @pl.when(c0 + b + ND < NCHUNK)
            def _():
                wait_idx(c0 + b + ND, b)
                fire_gather(b)
        return 0

    lax.fori_loop(0, NCHUNK // ND, _quad, 0)

    for b in range(NCHUNK % ND):
        wait_gather(b)
        fire_scatter(b)
    for b in range(NCHUNK % ND):
        wait_scatter(b)

    plsc.subcore_barrier()

    for j in range(ZROWS // DCH):
        start = pl.multiple_of(row0 + j * DCH, DCH)

        @pl.when(start < V)
        def _():
            pltpu.async_copy(agg_sh.at[pl.ds(start, DCH)],
                             out_hbm.at[pl.ds(c * V + start, DCH)], isems[0])

    for j in range(ZROWS // DCH):
        start = pl.multiple_of(row0 + j * DCH, DCH)

        @pl.when(start < V)
        def _():
            pltpu.make_async_copy(agg_sh.at[pl.ds(start, DCH)],
                                  out_hbm.at[pl.ds(c * V + start, DCH)], isems[0]).wait()


@functools.cache
def _sc_scatter_kernel():
    return pl.kernel(
        _sc_body,
        out_type=jax.ShapeDtypeStruct((2 * V, HID), jnp.float32),
        mesh=plsc.VectorSubcoreMesh(core_axis_name="c", subcore_axis_name="s"),
        scratch_types=[
            pltpu.VMEM_SHARED((V, HID), jnp.float32),
            [pltpu.VMEM((2, CH), jnp.int32) for _ in range(ND)],
            [pltpu.VMEM((CH, HID), jnp.float32) for _ in range(ND)],
            [pltpu.VMEM((CH, HID), jnp.float32) for _ in range(ND)],
            [pltpu.SemaphoreType.DMA for _ in range(ND)],
            [pltpu.SemaphoreType.DMA for _ in range(ND)],
            [pltpu.SemaphoreType.DMA for _ in range(ND)],
        ],
    )


def _sc_scatter(nbr, idx_r):
    return _sc_scatter_kernel()(nbr, idx_r)


VB = 2000
NVB = V // VB


def _bilinear_onehot(v):
    gx = jnp.clip((v[:, 0:1] + 1.0) * 6.5, 0.0, 13.0)
    gy = jnp.clip((v[:, 1:2] + 1.0) * 6.5, 0.0, 13.0)
    x0 = jnp.floor(gx)
    y0 = jnp.floor(gy)
    x1 = jnp.minimum(x0 + 1.0, 13.0)
    y1 = jnp.minimum(y0 + 1.0, 13.0)
    wx = gx - x0
    wy = gy - y0
    pio = lax.broadcasted_iota(jnp.int32, (VB, HW), 1)
    z = jnp.zeros((VB, HW), jnp.float32)

    def corner(yi, xi, w):
        p = (yi * 14.0 + xi).astype(jnp.int32)
        return jnp.where(pio == p, w, z)

    return (corner(y0, x0, (1 - wx) * (1 - wy)) +
            corner(y0, x1, wx * (1 - wy)) +
            corner(y1, x0, (1 - wx) * wy) +
            corner(y1, x1, wx * wy))


def _head_body_vf(v_ref, x2_ref, bwT_ref, bb_ref, vf_ref,
                  w0f_ref, w0i_ref, w0v_ref, b0_ref,
                  w1f_ref, w1i_ref, w1v_ref, b1_ref,
                  out_ref, nbr_ref):
    v = v_ref[...]
    A = _bilinear_onehot(v)
    imgW = _f32dot(x2_ref[...], bwT_ref[...])
    img = jnp.maximum(_f32dot(A, imgW) + bb_ref[...], 0.0)
    vf = vf_ref[...]
    out_ref[...] = (_f32dot(vf, w0f_ref[...]) + _f32dot(img, w0i_ref[...]) +
                    _f32dot(v, w0v_ref[...]) + b0_ref[...])
    nbr_ref[...] = (_f32dot(vf, w1f_ref[...]) + _f32dot(img, w1i_ref[...]) +
                    _f32dot(v, w1v_ref[...]) + b1_ref[...])


def _head_body_novf(v_ref, x2_ref, bwT_ref, bb_ref,
                    w0i_ref, w0v_ref, b0_ref,
                    w1i_ref, w1v_ref, b1_ref,
                    out_ref, nbr_ref):
    v = v_ref[...]
    A = _bilinear_onehot(v)
    imgW = _f32dot(x2_ref[...], bwT_ref[...])
    img = jnp.maximum(_f32dot(A, imgW) + bb_ref[...], 0.0)
    out_ref[...] = (_f32dot(img, w0i_ref[...]) +
                    _f32dot(v, w0v_ref[...]) + b0_ref[...])
    nbr_ref[...] = (_f32dot(img, w1i_ref[...]) +
                    _f32dot(v, w1v_ref[...]) + b1_ref[...])


def _mid_body(o_ref, agg_ref, v_ref,
              w0a_ref, w0v_ref, b0_ref, w1a_ref, w1v_ref, b1_ref,
              out_ref, nbr_ref):
    agg = agg_ref[0] + agg_ref[1]
    nopos = jnp.maximum(o_ref[...] + agg, 0.0)
    v = v_ref[...]
    out_ref[...] = (_f32dot(nopos, w0a_ref[...]) +
                    _f32dot(v, w0v_ref[...]) + b0_ref[...])
    nbr_ref[...] = (_f32dot(nopos, w1a_ref[...]) +
                    _f32dot(v, w1v_ref[...]) + b1_ref[...])


def _tail_body(o_ref, agg_ref, v_ref, offa_ref, offv_ref, offb_ref,
               vout_ref, nopos_ref):
    agg = agg_ref[0] + agg_ref[1]
    nopos = jnp.maximum(o_ref[...] + agg, 0.0)
    v = v_ref[...]
    deform = jnp.tanh(_f32dot(nopos, offa_ref[...]) +
                      _f32dot(v, offv_ref[...]) + offb_ref[...])
    vout_ref[...] = v + deform
    nopos_ref[...] = nopos


_vh = jax.ShapeDtypeStruct((V, HID), jnp.float32)
_v3 = jax.ShapeDtypeStruct((V, 3), jnp.float32)


def _rows(shape):
    return pl.BlockSpec((VB,) + shape[1:], lambda b: (b,) + (0,) * (len(shape) - 1))


def _const(shape):
    return pl.BlockSpec(shape, lambda b: (0,) * len(shape))


_AGG = pl.BlockSpec((2, VB, HID), lambda b: (0, b, 0))

_head_vf = pl.pallas_call(
    _head_body_vf, grid=(NVB,), out_shape=[_vh, _vh],
    in_specs=[_rows((V, 3)), _const((HW, C)), _const((C, HID)), _const((1, HID)),
              _rows((V, HID)),
              _const((HID, HID)), _const((HID, HID)), _const((3, HID)), _const((1, HID)),
              _const((HID, HID)), _const((HID, HID)), _const((3, HID)), _const((1, HID))],
    out_specs=[_rows((V, HID)), _rows((V, HID))])

_head_novf = pl.pallas_call(
    _head_body_novf, grid=(NVB,), out_shape=[_vh, _vh],
    in_specs=[_rows((V, 3)), _const((HW, C)), _const((C, HID)), _const((1, HID)),
              _const((HID, HID)), _const((3, HID)), _const((1, HID)),
              _const((HID, HID)), _const((3, HID)), _const((1, HID))],
    out_specs=[_rows((V, HID)), _rows((V, HID))])

_mid = pl.pallas_call(
    _mid_body, grid=(NVB,), out_shape=[_vh, _vh],
    in_specs=[_rows((V, HID)), _AGG, _rows((V, 3)),
              _const((HID, HID)), _const((3, HID)), _const((1, HID)),
              _const((HID, HID)), _const((3, HID)), _const((1, HID))],
    out_specs=[_rows((V, HID)), _rows((V, HID))])

_tail = pl.pallas_call(
    _tail_body, grid=(NVB,), out_shape=[_v3, _vh],
    in_specs=[_rows((V, HID)), _AGG, _rows((V, 3)),
              _const((HID, 3)), _const((3, 3)), _const((1, 3))],
    out_specs=[_rows((V, 3)), _rows((V, HID))])


def kernel(x, verts, edges, params):
    src = edges[:, 0]
    dst = edges[:, 1]
    idx_r = jnp.stack([src.reshape(NW, NCHUNK, CH),
                       dst.reshape(NW, NCHUNK, CH)], axis=2)
    x2 = jnp.transpose(x[0].reshape(C, HW))
    stage_verts = []
    vert_feats = None
    for sp in params:
        bwT = jnp.transpose(sp["bneck_W"])
        bb = sp["bneck_b"][None, :]
        g0 = sp["gconvs"][0]
        w0W, w0b, w1W, w1b = g0
        if vert_feats is None:
            w0i = jnp.transpose(w0W[:, :HID]); w0v = jnp.transpose(w0W[:, HID:])
            w1i = jnp.transpose(w1W[:, :HID]); w1v = jnp.transpose(w1W[:, HID:])
            out, nbr = _head_novf(verts, x2, bwT, bb,
                                  w0i, w0v, w0b[None, :],
                                  w1i, w1v, w1b[None, :])
        else:
            w0f = jnp.transpose(w0W[:, :HID])
            w0i = jnp.transpose(w0W[:, HID:2 * HID])
            w0v = jnp.transpose(w0W[:, 2 * HID:])
            w1f = jnp.transpose(w1W[:, :HID])
            w1i = jnp.transpose(w1W[:, HID:2 * HID])
            w1v = jnp.transpose(w1W[:, 2 * HID:])
            out, nbr = _head_vf(verts, x2, bwT, bb, vert_feats,
                                w0f, w0i, w0v, w0b[None, :],
                                w1f, w1i, w1v, w1b[None, :])
        for i in (1, 2):
            agg2 = _sc_scatter(nbr, idx_r).reshape(2, V, HID)
            g = sp["gconvs"][i]
            w0W, w0b, w1W, w1b = g
            out, nbr = _mid(out, agg2, verts,
                            jnp.transpose(w0W[:, :HID]),
                            jnp.transpose(w0W[:, HID:]), w0b[None, :],
                            jnp.transpose(w1W[:, :HID]),
                            jnp.transpose(w1W[:, HID:]), w1b[None, :])
        agg2 = _sc_scatter(nbr, idx_r).reshape(2, V, HID)
        offW, offb = sp["off_W"], sp["off_b"]
        verts, vert_feats = _tail(out, agg2, verts,
                                  jnp.transpose(offW[:, :HID]),
                                  jnp.transpose(offW[:, HID:]),
                                  offb[None, :])
        stage_verts.append(verts)
    return jnp.stack(stage_verts)


# speedup vs baseline: 1.1546x; 1.1546x over previous
import functools

import jax
import jax.numpy as jnp
from jax import lax
from jax.experimental import pallas as pl
from jax.experimental.pallas import tpu as pltpu
from jax.experimental.pallas import tpu_sc as plsc

V = 10000
E = 320000
C = 256
HW = 196
HID = 128

NC = 2
NS = 16
NW = NC * NS
EPW = E // NW
CH = 80
NCHUNK = EPW // CH
ZROWS = 640


def _f32dot(a, b):
    return jnp.dot(a, b, preferred_element_type=jnp.float32)


def _sc_body(nbr_hbm, src_hbm, dst_hbm, out_hbm,
             agg_sh, idxA0, idxB0, idxA1, idxB1,
             rowsA0, rowsB0, rowsA1, rowsB1,
             isem0, isem1, gsem0, gsem1, ssem0, ssem1):
    c = lax.axis_index("c")
    s = lax.axis_index("s")
    wid = s * NC + c
    ebase = wid * EPW

    def fire_idx(cidx, idxA, idxB, isem):
        b = ebase + cidx * CH
        pltpu.async_copy(src_hbm.at[pl.ds(b, CH)], idxA, isem)
        pltpu.async_copy(dst_hbm.at[pl.ds(b, CH)], idxB, isem)

    def wait_idx(cidx, idxA, idxB, isem):
        b = ebase + cidx * CH
        pltpu.make_async_copy(src_hbm.at[pl.ds(b, CH)], idxA, isem).wait()
        pltpu.make_async_copy(dst_hbm.at[pl.ds(b, CH)], idxB, isem).wait()

    def fire_gather(idxA, idxB, rowsA, rowsB, gsem):
        pltpu.async_copy(nbr_hbm.at[idxA], rowsA, gsem)
        pltpu.async_copy(nbr_hbm.at[idxB], rowsB, gsem)

    def wait_gather(idxA, idxB, rowsA, rowsB, gsem):
        pltpu.make_async_copy(nbr_hbm.at[idxA], rowsA, gsem).wait()
        pltpu.make_async_copy(nbr_hbm.at[idxB], rowsB, gsem).wait()

    def fire_scatter(idxA, idxB, rowsA, rowsB, ssem):
        pltpu.async_copy(rowsA, agg_sh.at[idxB], ssem, add=True)
        pltpu.async_copy(rowsB, agg_sh.at[idxA], ssem, add=True)

    def wait_scatter(idxA, idxB, rowsA, rowsB, ssem):
        pltpu.make_async_copy(rowsA, agg_sh.at[idxB], ssem).wait()
        pltpu.make_async_copy(rowsB, agg_sh.at[idxA], ssem).wait()

    zvec = jnp.zeros((16,), jnp.float32)

    def _zero_row(i, _):
        for k in range(HID // 16):
            rowsA0[i, pl.ds(k * 16, 16)] = zvec
        return 0

    lax.fori_loop(0, CH, _zero_row, 0)

    row0 = s * ZROWS
    for j in range(ZROWS // CH):
        start = row0 + j * CH

        @pl.when(start < V)
        def _():
            pltpu.async_copy(rowsA0, agg_sh.at[pl.ds(start, CH)], isem0)

    for j in range(ZROWS // CH):
        start = row0 + j * CH

        @pl.when(start < V)
        def _():
            pltpu.make_async_copy(rowsA0, agg_sh.at[pl.ds(start, CH)], isem0).wait()

    pltpu.sync_copy(src_hbm.at[pl.ds(ebase, CH)], idxA0)
    pltpu.sync_copy(dst_hbm.at[pl.ds(ebase, CH)], idxB0)
    fire_gather(idxA0, idxB0, rowsA0, rowsB0, gsem0)

    plsc.subcore_barrier()

    def _pair(j, _):
        c0 = 2 * j
        c1 = c0 + 1

        @pl.when(j > 0)
        def _():
            wait_scatter(idxA1, idxB1, rowsA1, rowsB1, ssem1)

        fire_idx(c1, idxA1, idxB1, isem1)
        wait_gather(idxA0, idxB0, rowsA0, rowsB0, gsem0)
        fire_scatter(idxA0, idxB0, rowsA0, rowsB0, ssem0)
        wait_idx(c1, idxA1, idxB1, isem1)
        fire_gather(idxA1, idxB1, rowsA1, rowsB1, gsem1)
        wait_scatter(idxA0, idxB0, rowsA0, rowsB0, ssem0)
        fire_idx(c0 + 2, idxA0, idxB0, isem0)
        wait_gather(idxA1, idxB1, rowsA1, rowsB1, gsem1)
        fire_scatter(idxA1, idxB1, rowsA1, rowsB1, ssem1)
        wait_idx(c0 + 2, idxA0, idxB0, isem0)
        fire_gather(idxA0, idxB0, rowsA0, rowsB0, gsem0)
        return 0

    lax.fori_loop(0, NCHUNK // 2, _pair, 0)

    wait_scatter(idxA1, idxB1, rowsA1, rowsB1, ssem1)
    wait_gather(idxA0, idxB0, rowsA0, rowsB0, gsem0)
    fire_scatter(idxA0, idxB0, rowsA0, rowsB0, ssem0)
    wait_scatter(idxA0, idxB0, rowsA0, rowsB0, ssem0)

    plsc.subcore_barrier()

    for j in range(ZROWS // CH):
        start = row0 + j * CH

        @pl.when(start < V)
        def _():
            pltpu.async_copy(agg_sh.at[pl.ds(start, CH)],
                             out_hbm.at[pl.ds(c * V + start, CH)], isem0)

    for j in range(ZROWS // CH):
        start = row0 + j * CH

        @pl.when(start < V)
        def _():
            pltpu.make_async_copy(agg_sh.at[pl.ds(start, CH)],
                                  out_hbm.at[pl.ds(c * V + start, CH)], isem0).wait()


@functools.cache
def _sc_scatter_kernel():
    return pl.kernel(
        _sc_body,
        out_type=jax.ShapeDtypeStruct((2 * V, HID), jnp.float32),
        mesh=plsc.VectorSubcoreMesh(core_axis_name="c", subcore_axis_name="s"),
        scratch_types=[
            pltpu.VMEM_SHARED((V, HID), jnp.float32),
            pltpu.VMEM((CH,), jnp.int32),
            pltpu.VMEM((CH,), jnp.int32),
            pltpu.VMEM((CH,), jnp.int32),
            pltpu.VMEM((CH,), jnp.int32),
            pltpu.VMEM((CH, HID), jnp.float32),
            pltpu.VMEM((CH, HID), jnp.float32),
            pltpu.VMEM((CH, HID), jnp.float32),
            pltpu.VMEM((CH, HID), jnp.float32),
            pltpu.SemaphoreType.DMA,
            pltpu.SemaphoreType.DMA,
            pltpu.SemaphoreType.DMA,
            pltpu.SemaphoreType.DMA,
            pltpu.SemaphoreType.DMA,
            pltpu.SemaphoreType.DMA,
        ],
    )


def _sc_scatter(nbr, src, dst):
    return _sc_scatter_kernel()(nbr, src, dst)


VB = 2000
NVB = V // VB


def _bilinear_onehot(v):
    gx = jnp.clip((v[:, 0:1] + 1.0) * 6.5, 0.0, 13.0)
    gy = jnp.clip((v[:, 1:2] + 1.0) * 6.5, 0.0, 13.0)
    x0 = jnp.floor(gx)
    y0 = jnp.floor(gy)
    x1 = jnp.minimum(x0 + 1.0, 13.0)
    y1 = jnp.minimum(y0 + 1.0, 13.0)
    wx = gx - x0
    wy = gy - y0
    pio = lax.broadcasted_iota(jnp.int32, (VB, HW), 1)
    z = jnp.zeros((VB, HW), jnp.float32)

    def corner(yi, xi, w):
        p = (yi * 14.0 + xi).astype(jnp.int32)
        return jnp.where(pio == p, w, z)

    return (corner(y0, x0, (1 - wx) * (1 - wy)) +
            corner(y0, x1, wx * (1 - wy)) +
            corner(y1, x0, (1 - wx) * wy) +
            corner(y1, x1, wx * wy))


def _head_body_vf(v_ref, x2_ref, bwT_ref, bb_ref, vf_ref,
                  w0f_ref, w0i_ref, w0v_ref, b0_ref,
                  w1f_ref, w1i_ref, w1v_ref, b1_ref,
                  out_ref, nbr_ref):
    v = v_ref[...]
    A = _bilinear_onehot(v)
    imgW = _f32dot(x2_ref[...], bwT_ref[...])
    img = jnp.maximum(_f32dot(A, imgW) + bb_ref[...], 0.0)
    vf = vf_ref[...]
    out_ref[...] = (_f32dot(vf, w0f_ref[...]) + _f32dot(img, w0i_ref[...]) +
                    _f32dot(v, w0v_ref[...]) + b0_ref[...])
    nbr_ref[...] = (_f32dot(vf, w1f_ref[...]) + _f32dot(img, w1i_ref[...]) +
                    _f32dot(v, w1v_ref[...]) + b1_ref[...])


def _head_body_novf(v_ref, x2_ref, bwT_ref, bb_ref,
                    w0i_ref, w0v_ref, b0_ref,
                    w1i_ref, w1v_ref, b1_ref,
                    out_ref, nbr_ref):
    v = v_ref[...]
    A = _bilinear_onehot(v)
    imgW = _f32dot(x2_ref[...], bwT_ref[...])
    img = jnp.maximum(_f32dot(A, imgW) + bb_ref[...], 0.0)
    out_ref[...] = (_f32dot(img, w0i_ref[...]) +
                    _f32dot(v, w0v_ref[...]) + b0_ref[...])
    nbr_ref[...] = (_f32dot(img, w1i_ref[...]) +
                    _f32dot(v, w1v_ref[...]) + b1_ref[...])


def _mid_body(o_ref, agg_ref, v_ref,
              w0a_ref, w0v_ref, b0_ref, w1a_ref, w1v_ref, b1_ref,
              out_ref, nbr_ref):
    agg = agg_ref[0] + agg_ref[1]
    nopos = jnp.maximum(o_ref[...] + agg, 0.0)
    v = v_ref[...]
    out_ref[...] = (_f32dot(nopos, w0a_ref[...]) +
                    _f32dot(v, w0v_ref[...]) + b0_ref[...])
    nbr_ref[...] = (_f32dot(nopos, w1a_ref[...]) +
                    _f32dot(v, w1v_ref[...]) + b1_ref[...])


def _tail_body(o_ref, agg_ref, v_ref, offa_ref, offv_ref, offb_ref,
               vout_ref, nopos_ref):
    agg = agg_ref[0] + agg_ref[1]
    nopos = jnp.maximum(o_ref[...] + agg, 0.0)
    v = v_ref[...]
    deform = jnp.tanh(_f32dot(nopos, offa_ref[...]) +
                      _f32dot(v, offv_ref[...]) + offb_ref[...])
    vout_ref[...] = v + deform
    nopos_ref[...] = nopos


_vh = jax.ShapeDtypeStruct((V, HID), jnp.float32)
_v3 = jax.ShapeDtypeStruct((V, 3), jnp.float32)


def _rows(shape):
    return pl.BlockSpec((VB,) + shape[1:], lambda b: (b,) + (0,) * (len(shape) - 1))


def _const(shape):
    return pl.BlockSpec(shape, lambda b: (0,) * len(shape))


_AGG = pl.BlockSpec((2, VB, HID), lambda b: (0, b, 0))

_head_vf = pl.pallas_call(
    _head_body_vf, grid=(NVB,), out_shape=[_vh, _vh],
    in_specs=[_rows((V, 3)), _const((HW, C)), _const((C, HID)), _const((1, HID)),
              _rows((V, HID)),
              _const((HID, HID)), _const((HID, HID)), _const((3, HID)), _const((1, HID)),
              _const((HID, HID)), _const((HID, HID)), _const((3, HID)), _const((1, HID))],
    out_specs=[_rows((V, HID)), _rows((V, HID))])

_head_novf = pl.pallas_call(
    _head_body_novf, grid=(NVB,), out_shape=[_vh, _vh],
    in_specs=[_rows((V, 3)), _const((HW, C)), _const((C, HID)), _const((1, HID)),
              _const((HID, HID)), _const((3, HID)), _const((1, HID)),
              _const((HID, HID)), _const((3, HID)), _const((1, HID))],
    out_specs=[_rows((V, HID)), _rows((V, HID))])

_mid = pl.pallas_call(
    _mid_body, grid=(NVB,), out_shape=[_vh, _vh],
    in_specs=[_rows((V, HID)), _AGG, _rows((V, 3)),
              _const((HID, HID)), _const((3, HID)), _const((1, HID)),
              _const((HID, HID)), _const((3, HID)), _const((1, HID))],
    out_specs=[_rows((V, HID)), _rows((V, HID))])

_tail = pl.pallas_call(
    _tail_body, grid=(NVB,), out_shape=[_v3, _vh],
    in_specs=[_rows((V, HID)), _AGG, _rows((V, 3)),
              _const((HID, 3)), _const((3, 3)), _const((1, 3))],
    out_specs=[_rows((V, 3)), _rows((V, HID))])


def kernel(x, verts, edges, params):
    src = edges[:, 0]
    dst = edges[:, 1]
    x2 = jnp.transpose(x[0].reshape(C, HW))
    stage_verts = []
    vert_feats = None
    for sp in params:
        bwT = jnp.transpose(sp["bneck_W"])
        bb = sp["bneck_b"][None, :]
        g0 = sp["gconvs"][0]
        w0W, w0b, w1W, w1b = g0
        if vert_feats is None:
            w0i = jnp.transpose(w0W[:, :HID]); w0v = jnp.transpose(w0W[:, HID:])
            w1i = jnp.transpose(w1W[:, :HID]); w1v = jnp.transpose(w1W[:, HID:])
            out, nbr = _head_novf(verts, x2, bwT, bb,
                                  w0i, w0v, w0b[None, :],
                                  w1i, w1v, w1b[None, :])
        else:
            w0f = jnp.transpose(w0W[:, :HID])
            w0i = jnp.transpose(w0W[:, HID:2 * HID])
            w0v = jnp.transpose(w0W[:, 2 * HID:])
            w1f = jnp.transpose(w1W[:, :HID])
            w1i = jnp.transpose(w1W[:, HID:2 * HID])
            w1v = jnp.transpose(w1W[:, 2 * HID:])
            out, nbr = _head_vf(verts, x2, bwT, bb, vert_feats,
                                w0f, w0i, w0v, w0b[None, :],
                                w1f, w1i, w1v, w1b[None, :])
        for i in (1, 2):
            agg2 = _sc_scatter(nbr, src, dst).reshape(2, V, HID)
            g = sp["gconvs"][i]
            w0W, w0b, w1W, w1b = g
            out, nbr = _mid(out, agg2, verts,
                            jnp.transpose(w0W[:, :HID]),
                            jnp.transpose(w0W[:, HID:]), w0b[None, :],
                            jnp.transpose(w1W[:, :HID]),
                            jnp.transpose(w1W[:, HID:]), w1b[None, :])
        agg2 = _sc_scatter(nbr, src, dst).reshape(2, V, HID)
        offW, offb = sp["off_W"], sp["off_b"]
        verts, vert_feats = _tail(out, agg2, verts,
                                  jnp.transpose(offW[:, :HID]),
                                  jnp.transpose(offW[:, HID:]),
                                  offb[None, :])
        stage_verts.append(verts)
    return jnp.stack(stage_verts)
